# baseline (speedup 1.0000x reference)
import jax
import jax.numpy as jnp
from jax.experimental import pallas as pl


def _body(idx_ref, x_ref, out_ref):
    n = x_ref.shape[1]
    idx = idx_ref[0, :]
    onehot = (jax.lax.broadcasted_iota(jnp.int32, (n, idx.shape[0]), 0)
              == idx[None, :]).astype(jnp.float32)
    out_ref[...] = jnp.dot(x_ref[...], onehot,
                           preferred_element_type=jnp.float32,
                           precision=jax.lax.Precision.DEFAULT)


def kernel(input, indices):
    B, S, N = input.shape
    K = indices.shape[0]
    R = B * S
    x = input.reshape(R, N)
    idx2 = indices.reshape(1, K)
    BR = 2048
    out = pl.pallas_call(
        _body,
        grid=(R // BR,),
        in_specs=[
            pl.BlockSpec((1, K), lambda i: (0, 0)),
            pl.BlockSpec((BR, N), lambda i: (i, 0)),
        ],
        out_specs=pl.BlockSpec((BR, K), lambda i: (i, 0)),
        out_shape=jax.ShapeDtypeStruct((R, K), x.dtype),
    )(idx2, x)
    return out.reshape(B, S, K)
